# Initial kernel scaffold; baseline (speedup 1.0000x reference)
import jax
import jax.numpy as jnp
from jax.experimental import pallas as pl


def kernel(x):
    raise NotImplementedError("write your pallas kernel here")


# speedup vs baseline: 1.5221x; 1.5221x over previous
import functools

import jax
import jax.numpy as jnp
from jax import lax
from jax.experimental import pallas as pl
from jax.experimental.pallas import tpu as pltpu
from jax.experimental.pallas import tpu_sc as plsc

SEG_NUM_ALL = 16
NUM_FPS = 64
N = 4096
BS = 8
L = 16
NCHUNKS = N // L
SEG_PER_TILE = 4
NTILES = 32
NEG = -1e10
BIG_IDX = 1 << 30


def _lane_shuffle(v, idx):
    dnums = lax.GatherDimensionNumbers(
        offset_dims=(), collapsed_slice_dims=(0,), start_index_map=(0,)
    )
    return lax.gather(
        v, idx[:, None], dimension_numbers=dnums, slice_sizes=(1,),
        mode=lax.GatherScatterMode.PROMISE_IN_BOUNDS,
    )


def _allreduce(v, op):
    iota = lax.iota(jnp.int32, L)
    for sh in (8, 4, 2, 1):
        v = op(v, _lane_shuffle(v, iota ^ sh))
    return v


def kernel(x):
    pts = jnp.transpose(x[..., :3], (2, 0, 1)).reshape(-1)
    labels = x[..., 3].astype(jnp.int32).reshape(-1)

    mesh = plsc.VectorSubcoreMesh(core_axis_name="c", subcore_axis_name="s")

    @functools.partial(
        pl.kernel,
        out_type=(
            jax.ShapeDtypeStruct((3 * NTILES * SEG_PER_TILE * NUM_FPS,), jnp.float32),
            jax.ShapeDtypeStruct((NTILES * L,), jnp.int32),
        ),
        mesh=mesh,
        compiler_params=pltpu.CompilerParams(needs_layout_passes=False),
        scratch_types=[
            pltpu.VMEM((N,), jnp.float32),
            pltpu.VMEM((N,), jnp.float32),
            pltpu.VMEM((N,), jnp.float32),
            pltpu.VMEM((N,), jnp.int32),
            pltpu.VMEM((N,), jnp.float32),
            pltpu.VMEM((N,), jnp.float32),
            pltpu.VMEM((N,), jnp.float32),
            pltpu.VMEM((N,), jnp.float32),
            pltpu.VMEM((SEG_PER_TILE * NUM_FPS,), jnp.float32),
            pltpu.VMEM((SEG_PER_TILE * NUM_FPS,), jnp.float32),
            pltpu.VMEM((SEG_PER_TILE * NUM_FPS,), jnp.float32),
            pltpu.VMEM((L,), jnp.int32),
        ],
    )
    def run(pts_hbm, lab_hbm, gp_hbm, cnt_hbm,
            pxv, pyv, pzv, labv, d0, d1, d2, d3, ox, oy, oz, cntv):
        wid = lax.axis_index("s") * 2 + lax.axis_index("c")
        b = wid // 4
        g = wid % 4
        distv = (d0, d1, d2, d3)
        outv = (ox, oy, oz)
        planes = (pxv, pyv, pzv)

        for d in range(3):
            pltpu.sync_copy(pts_hbm.at[pl.ds((d * BS + b) * N, N)], planes[d])
        pltpu.sync_copy(lab_hbm.at[pl.ds(b * N, N)], labv)

        iota = lax.iota(jnp.int32, L)
        cnt_vec = jnp.zeros((L,), jnp.int32)

        for k in range(SEG_PER_TILE):
            s = g * SEG_PER_TILE + k
            dk = distv[k]

            def init_chunk(j, carry, dk=dk, s=s):
                acc, fidx = carry
                lab_c = labv[pl.ds(j * L, L)]
                m = lab_c == s
                acc = acc + jnp.where(m, 1, 0).astype(jnp.int32)
                lin = j * L + iota
                fidx = jnp.minimum(fidx, jnp.where(m, lin, BIG_IDX))
                dk[pl.ds(j * L, L)] = jnp.where(m, jnp.float32(1e10),
                                                jnp.float32(NEG))
                return acc, fidx

            acc0 = jnp.zeros((L,), jnp.int32)
            fidx0 = jnp.full((L,), BIG_IDX, jnp.int32)
            acc, fidx = lax.fori_loop(0, NCHUNKS, init_chunk, (acc0, fidx0))
            countv = _allreduce(acc, jnp.add)
            firstv = _allreduce(fidx, jnp.minimum)
            firstv = jnp.where(firstv >= N, 0, firstv)

            def step(t, carry, dk=dk):
                curv, sel = carry
                lane = t % L
                w = t // L
                lanemask = iota == lane
                sel = tuple(
                    jnp.where((w == wi) & lanemask, curv, sel[wi])
                    for wi in range(4)
                )
                cx = plsc.load_gather(pxv, [curv])
                cy = plsc.load_gather(pyv, [curv])
                cz = plsc.load_gather(pzv, [curv])

                def chunk(j, c2, dk=dk):
                    bv, bi = c2
                    px = pxv[pl.ds(j * L, L)]
                    py = pyv[pl.ds(j * L, L)]
                    pz = pzv[pl.ds(j * L, L)]
                    dist = dk[pl.ds(j * L, L)]
                    dx = px - cx
                    dy = py - cy
                    dz = pz - cz
                    dd = dx * dx + dy * dy + dz * dz
                    nd = jnp.minimum(dist, dd)
                    dk[pl.ds(j * L, L)] = nd
                    pred = nd > bv
                    bv = jnp.where(pred, nd, bv)
                    bi = jnp.where(pred, j * L + iota, bi)
                    return bv, bi

                bv0 = jnp.full((L,), jnp.float32(-3e38))
                bi0 = jnp.zeros((L,), jnp.int32)
                bv, bi = lax.fori_loop(0, NCHUNKS, chunk, (bv0, bi0))
                maxv = _allreduce(bv, jnp.maximum)
                cand = jnp.where(bv == maxv, bi, BIG_IDX)
                nxtv = _allreduce(cand, jnp.minimum)
                nxtv = jnp.where(nxtv >= N, 0, nxtv)
                return nxtv, sel

            sel0 = tuple(jnp.zeros((L,), jnp.int32) for _ in range(4))
            _, sel = lax.fori_loop(0, NUM_FPS, step, (firstv, sel0))

            multv = jnp.where(countv >= NUM_FPS, jnp.float32(1.0),
                              jnp.float32(0.0))
            for wi in range(4):
                for d in range(3):
                    vec = plsc.load_gather(planes[d], [sel[wi]]) * multv
                    outv[d][pl.ds(k * NUM_FPS + wi * L, L)] = vec

            cnt_vec = jnp.where(iota == k, countv, cnt_vec)

        blk = SEG_PER_TILE * NUM_FPS
        for d in range(3):
            pltpu.sync_copy(outv[d],
                            gp_hbm.at[pl.ds((d * NTILES + wid) * blk, blk)])
        cntv[...] = cnt_vec
        pltpu.sync_copy(cntv, cnt_hbm.at[pl.ds(wid * L, L)])

    gp_raw, cnt_raw = run(pts, labels)

    counts = cnt_raw.reshape(NTILES, L)[:, :SEG_PER_TILE].reshape(BS, SEG_NUM_ALL)
    masks = counts >= NUM_FPS
    gp = (
        gp_raw.reshape(3, BS, 4, SEG_PER_TILE, NUM_FPS)
        .transpose(1, 2, 3, 4, 0)
        .reshape(BS, SEG_NUM_ALL, NUM_FPS, 3)
    )
    return gp, masks


# speedup vs baseline: 12.1181x; 7.9614x over previous
import functools

import jax
import jax.numpy as jnp
from jax import lax
from jax.experimental import pallas as pl
from jax.experimental.pallas import tpu as pltpu
from jax.experimental.pallas import tpu_sc as plsc

SEG_NUM_ALL = 16
NUM_FPS = 64
N = 4096
BS = 8
L = 16
NCHUNKS = N // L
SEG_PER_TILE = 4
NTILES = 32
NEG = -1e10
BIG_IDX = 1 << 30


def _lane_shuffle(v, idx):
    dnums = lax.GatherDimensionNumbers(
        offset_dims=(), collapsed_slice_dims=(0,), start_index_map=(0,)
    )
    return lax.gather(
        v, idx[:, None], dimension_numbers=dnums, slice_sizes=(1,),
        mode=lax.GatherScatterMode.PROMISE_IN_BOUNDS,
    )


def _allreduce(v, op):
    iota = lax.iota(jnp.int32, L)
    for sh in (8, 4, 2, 1):
        v = op(v, _lane_shuffle(v, iota ^ sh))
    return v


def kernel(x):
    pts = jnp.transpose(x[..., :3], (2, 0, 1)).reshape(-1)
    labels = x[..., 3].astype(jnp.int32).reshape(-1)

    mesh = plsc.VectorSubcoreMesh(core_axis_name="c", subcore_axis_name="s")

    @functools.partial(
        pl.kernel,
        out_type=(
            jax.ShapeDtypeStruct((3 * NTILES * SEG_PER_TILE * NUM_FPS,), jnp.float32),
            jax.ShapeDtypeStruct((NTILES * L,), jnp.int32),
        ),
        mesh=mesh,
        compiler_params=pltpu.CompilerParams(needs_layout_passes=False),
        scratch_types=[
            pltpu.VMEM((N,), jnp.float32),
            pltpu.VMEM((N,), jnp.float32),
            pltpu.VMEM((N,), jnp.float32),
            pltpu.VMEM((N,), jnp.int32),
            pltpu.VMEM((N,), jnp.float32), pltpu.VMEM((N,), jnp.float32),
            pltpu.VMEM((N,), jnp.float32), pltpu.VMEM((N,), jnp.float32),
            pltpu.VMEM((N,), jnp.float32), pltpu.VMEM((N,), jnp.float32),
            pltpu.VMEM((N,), jnp.float32), pltpu.VMEM((N,), jnp.float32),
            pltpu.VMEM((N,), jnp.float32), pltpu.VMEM((N,), jnp.float32),
            pltpu.VMEM((N,), jnp.float32), pltpu.VMEM((N,), jnp.float32),
            pltpu.VMEM((N,), jnp.float32), pltpu.VMEM((N,), jnp.float32),
            pltpu.VMEM((N,), jnp.float32), pltpu.VMEM((N,), jnp.float32),
            pltpu.VMEM((SEG_PER_TILE * NUM_FPS,), jnp.float32),
            pltpu.VMEM((SEG_PER_TILE * NUM_FPS,), jnp.float32),
            pltpu.VMEM((SEG_PER_TILE * NUM_FPS,), jnp.float32),
            pltpu.VMEM((L,), jnp.int32),
        ],
    )
    def run(pts_hbm, lab_hbm, gp_hbm, cnt_hbm,
            pxv, pyv, pzv, labv,
            cx0, cy0, cz0, cx1, cy1, cz1, cx2, cy2, cz2, cx3, cy3, cz3,
            d0, d1, d2, d3, ox, oy, oz, cntv):
        wid = lax.axis_index("s") * 2 + lax.axis_index("c")
        b = wid // 4
        g = wid % 4
        comp = ((cx0, cy0, cz0), (cx1, cy1, cz1),
                (cx2, cy2, cz2), (cx3, cy3, cz3))
        distv = (d0, d1, d2, d3)
        outv = (ox, oy, oz)
        planes = (pxv, pyv, pzv)

        for d in range(3):
            pltpu.sync_copy(pts_hbm.at[pl.ds((d * BS + b) * N, N)], planes[d])
        pltpu.sync_copy(lab_hbm.at[pl.ds(b * N, N)], labv)

        iota = lax.iota(jnp.int32, L)

        def compact_chunk(j, offs):
            lab_c = labv[pl.ds(j * L, L)]
            px = pxv[pl.ds(j * L, L)]
            py = pyv[pl.ds(j * L, L)]
            pz = pzv[pl.ds(j * L, L)]
            new_offs = []
            for k in range(SEG_PER_TILE):
                s = g * SEG_PER_TILE + k
                m = lab_c == s
                mi = jnp.where(m, 1, 0).astype(jnp.int32)
                pos = offs[k] + jnp.cumsum(mi) - 1
                plsc.store_scatter(comp[k][0], [pos], px, mask=m)
                plsc.store_scatter(comp[k][1], [pos], py, mask=m)
                plsc.store_scatter(comp[k][2], [pos], pz, mask=m)
                new_offs.append(offs[k] + plsc.all_reduce_population_count(m))
            return tuple(new_offs)

        offs0 = tuple(jnp.zeros((L,), jnp.int32) for _ in range(SEG_PER_TILE))
        offs = lax.fori_loop(0, NCHUNKS, compact_chunk, offs0)

        cnt_vec = jnp.zeros((L,), jnp.int32)

        for k in range(SEG_PER_TILE):
            ckx, cky, ckz = comp[k]
            dk = distv[k]
            countv = offs[k]
            count = jnp.max(countv)
            nch = (count + L - 1) // L

            def init_chunk(j, _, dk=dk, countv=countv):
                lin = j * L + iota
                dk[pl.ds(j * L, L)] = jnp.where(
                    lin < countv, jnp.float32(1e10), jnp.float32(NEG))
                return 0

            lax.fori_loop(0, nch, init_chunk, 0)

            def step(t, carry, ckx=ckx, cky=cky, ckz=ckz, dk=dk, nch=nch):
                curv, sel = carry
                lane = t % L
                w = t // L
                lanemask = iota == lane
                sel = tuple(
                    jnp.where((w == wi) & lanemask, curv, sel[wi])
                    for wi in range(4)
                )
                cx = plsc.load_gather(ckx, [curv])
                cy = plsc.load_gather(cky, [curv])
                cz = plsc.load_gather(ckz, [curv])

                def chunk(j, c2, ckx=ckx, cky=cky, ckz=ckz, dk=dk):
                    bv, bi = c2
                    px = ckx[pl.ds(j * L, L)]
                    py = cky[pl.ds(j * L, L)]
                    pz = ckz[pl.ds(j * L, L)]
                    dist = dk[pl.ds(j * L, L)]
                    dx = px - cx
                    dy = py - cy
                    dz = pz - cz
                    dd = dx * dx + dy * dy + dz * dz
                    nd = jnp.minimum(dist, dd)
                    dk[pl.ds(j * L, L)] = nd
                    pred = nd > bv
                    bv = jnp.where(pred, nd, bv)
                    bi = jnp.where(pred, j * L + iota, bi)
                    return bv, bi

                bv0 = jnp.full((L,), jnp.float32(-3e38))
                bi0 = jnp.zeros((L,), jnp.int32)
                bv, bi = lax.fori_loop(0, nch, chunk, (bv0, bi0))
                maxv = _allreduce(bv, jnp.maximum)
                cand = jnp.where(bv == maxv, bi, BIG_IDX)
                nxtv = _allreduce(cand, jnp.minimum)
                nxtv = jnp.where(nxtv >= N, 0, nxtv)
                return nxtv, sel

            sel0 = tuple(jnp.zeros((L,), jnp.int32) for _ in range(4))
            _, sel = lax.fori_loop(
                0, NUM_FPS, step, (jnp.zeros((L,), jnp.int32), sel0))

            multv = jnp.where(countv >= NUM_FPS, jnp.float32(1.0),
                              jnp.float32(0.0))
            for wi in range(4):
                vx = plsc.load_gather(ckx, [sel[wi]]) * multv
                vy = plsc.load_gather(cky, [sel[wi]]) * multv
                vz = plsc.load_gather(ckz, [sel[wi]]) * multv
                ox[pl.ds(k * NUM_FPS + wi * L, L)] = vx
                oy[pl.ds(k * NUM_FPS + wi * L, L)] = vy
                oz[pl.ds(k * NUM_FPS + wi * L, L)] = vz

            cnt_vec = jnp.where(iota == k, countv, cnt_vec)

        blk = SEG_PER_TILE * NUM_FPS
        for d in range(3):
            pltpu.sync_copy(outv[d],
                            gp_hbm.at[pl.ds((d * NTILES + wid) * blk, blk)])
        cntv[...] = cnt_vec
        pltpu.sync_copy(cntv, cnt_hbm.at[pl.ds(wid * L, L)])

    gp_raw, cnt_raw = run(pts, labels)

    counts = cnt_raw.reshape(NTILES, L)[:, :SEG_PER_TILE].reshape(BS, SEG_NUM_ALL)
    masks = counts >= NUM_FPS
    gp = (
        gp_raw.reshape(3, BS, 4, SEG_PER_TILE, NUM_FPS)
        .transpose(1, 2, 3, 4, 0)
        .reshape(BS, SEG_NUM_ALL, NUM_FPS, 3)
    )
    return gp, masks


# speedup vs baseline: 19.0499x; 1.5720x over previous
import functools

import jax
import jax.numpy as jnp
from jax import lax
from jax.experimental import pallas as pl
from jax.experimental.pallas import tpu as pltpu
from jax.experimental.pallas import tpu_sc as plsc

SEG_NUM_ALL = 16
NUM_FPS = 64
N = 4096
BS = 8
L = 16
NCHUNKS = N // L
SEG_PER_TILE = 4
NTILES = 32
NEG = -1e10
BIG_IDX = 1 << 30


def _lane_shuffle(v, idx):
    dnums = lax.GatherDimensionNumbers(
        offset_dims=(), collapsed_slice_dims=(0,), start_index_map=(0,)
    )
    return lax.gather(
        v, idx[:, None], dimension_numbers=dnums, slice_sizes=(1,),
        mode=lax.GatherScatterMode.PROMISE_IN_BOUNDS,
    )


def _allreduce(v, op):
    iota = lax.iota(jnp.int32, L)
    for sh in (8, 4, 2, 1):
        v = op(v, _lane_shuffle(v, iota ^ sh))
    return v


def kernel(x):
    pts = jnp.transpose(x[..., :3], (2, 0, 1)).reshape(-1)
    labels = x[..., 3].astype(jnp.int32).reshape(-1)

    mesh = plsc.VectorSubcoreMesh(core_axis_name="c", subcore_axis_name="s")

    @functools.partial(
        pl.kernel,
        out_type=(
            jax.ShapeDtypeStruct((3 * NTILES * SEG_PER_TILE * NUM_FPS,), jnp.float32),
            jax.ShapeDtypeStruct((NTILES * L,), jnp.int32),
        ),
        mesh=mesh,
        compiler_params=pltpu.CompilerParams(needs_layout_passes=False),
        scratch_types=[
            pltpu.VMEM((N,), jnp.float32),
            pltpu.VMEM((N,), jnp.float32),
            pltpu.VMEM((N,), jnp.float32),
            pltpu.VMEM((N,), jnp.int32),
            pltpu.VMEM((N,), jnp.float32), pltpu.VMEM((N,), jnp.float32),
            pltpu.VMEM((N,), jnp.float32), pltpu.VMEM((N,), jnp.float32),
            pltpu.VMEM((N,), jnp.float32), pltpu.VMEM((N,), jnp.float32),
            pltpu.VMEM((N,), jnp.float32), pltpu.VMEM((N,), jnp.float32),
            pltpu.VMEM((N,), jnp.float32), pltpu.VMEM((N,), jnp.float32),
            pltpu.VMEM((N,), jnp.float32), pltpu.VMEM((N,), jnp.float32),
            pltpu.VMEM((N,), jnp.float32), pltpu.VMEM((N,), jnp.float32),
            pltpu.VMEM((N,), jnp.float32), pltpu.VMEM((N,), jnp.float32),
            pltpu.VMEM((SEG_PER_TILE * NUM_FPS,), jnp.float32),
            pltpu.VMEM((SEG_PER_TILE * NUM_FPS,), jnp.float32),
            pltpu.VMEM((SEG_PER_TILE * NUM_FPS,), jnp.float32),
            pltpu.VMEM((L,), jnp.int32),
        ],
    )
    def run(pts_hbm, lab_hbm, gp_hbm, cnt_hbm,
            pxv, pyv, pzv, labv,
            cx0, cy0, cz0, cx1, cy1, cz1, cx2, cy2, cz2, cx3, cy3, cz3,
            d0, d1, d2, d3, ox, oy, oz, cntv):
        wid = lax.axis_index("s") * 2 + lax.axis_index("c")
        b = wid // 4
        g = wid % 4
        comp = ((cx0, cy0, cz0), (cx1, cy1, cz1),
                (cx2, cy2, cz2), (cx3, cy3, cz3))
        distv = (d0, d1, d2, d3)
        outv = (ox, oy, oz)
        planes = (pxv, pyv, pzv)

        for d in range(3):
            pltpu.sync_copy(pts_hbm.at[pl.ds((d * BS + b) * N, N)], planes[d])
        pltpu.sync_copy(lab_hbm.at[pl.ds(b * N, N)], labv)

        iota = lax.iota(jnp.int32, L)

        offs0 = tuple(jnp.zeros((L,), jnp.int32) for _ in range(SEG_PER_TILE))

        @plsc.parallel_loop(0, NCHUNKS, unroll=2, carry=offs0)
        def compact_chunk(j, offs):
            lab_c = labv[pl.ds(j * L, L)]
            px = pxv[pl.ds(j * L, L)]
            py = pyv[pl.ds(j * L, L)]
            pz = pzv[pl.ds(j * L, L)]
            new_offs = []
            for k in range(SEG_PER_TILE):
                s = g * SEG_PER_TILE + k
                m = lab_c == s
                mi = jnp.where(m, 1, 0).astype(jnp.int32)
                pos = offs[k] + jnp.cumsum(mi) - 1
                plsc.store_scatter(comp[k][0], [pos], px, mask=m)
                plsc.store_scatter(comp[k][1], [pos], py, mask=m)
                plsc.store_scatter(comp[k][2], [pos], pz, mask=m)
                new_offs.append(offs[k] + plsc.all_reduce_population_count(m))
            return tuple(new_offs)

        offs = compact_chunk

        cnt_vec = jnp.zeros((L,), jnp.int32)

        for k in range(SEG_PER_TILE):
            ckx, cky, ckz = comp[k]
            dk = distv[k]
            countv = offs[k]
            count = jnp.max(countv)
            nch = ((count + 4 * L - 1) // (4 * L)) * 4

            @plsc.parallel_loop(0, nch, unroll=4)
            def init_chunk(j, dk=dk, countv=countv):
                lin = j * L + iota
                dk[pl.ds(j * L, L)] = jnp.where(
                    lin < countv, jnp.float32(1e10), jnp.float32(NEG))

            def step(t, carry, ckx=ckx, cky=cky, ckz=ckz, dk=dk, nch=nch):
                curv, sel = carry
                lane = t % L
                w = t // L
                lanemask = iota == lane
                sel = tuple(
                    jnp.where((w == wi) & lanemask, curv, sel[wi])
                    for wi in range(4)
                )
                cx = plsc.load_gather(ckx, [curv])
                cy = plsc.load_gather(cky, [curv])
                cz = plsc.load_gather(ckz, [curv])

                bv0 = jnp.full((L,), jnp.float32(-3e38))
                bi0 = jnp.zeros((L,), jnp.int32)

                @plsc.parallel_loop(0, nch, unroll=4, carry=(bv0, bi0))
                def chunk(j, c2, ckx=ckx, cky=cky, ckz=ckz, dk=dk):
                    bv, bi = c2
                    px = ckx[pl.ds(j * L, L)]
                    py = cky[pl.ds(j * L, L)]
                    pz = ckz[pl.ds(j * L, L)]
                    dist = dk[pl.ds(j * L, L)]
                    dx = px - cx
                    dy = py - cy
                    dz = pz - cz
                    dd = dx * dx + dy * dy + dz * dz
                    nd = jnp.minimum(dist, dd)
                    dk[pl.ds(j * L, L)] = nd
                    pred = nd > bv
                    bv = jnp.where(pred, nd, bv)
                    bi = jnp.where(pred, j * L + iota, bi)
                    return bv, bi

                bv, bi = chunk
                maxv = _allreduce(bv, jnp.maximum)
                cand = jnp.where(bv == maxv, bi, BIG_IDX)
                nxtv = _allreduce(cand, jnp.minimum)
                nxtv = jnp.where(nxtv >= N, 0, nxtv)
                return nxtv, sel

            sel0 = tuple(jnp.zeros((L,), jnp.int32) for _ in range(4))
            _, sel = lax.fori_loop(
                0, NUM_FPS, step, (jnp.zeros((L,), jnp.int32), sel0))

            multv = jnp.where(countv >= NUM_FPS, jnp.float32(1.0),
                              jnp.float32(0.0))
            for wi in range(4):
                vx = plsc.load_gather(ckx, [sel[wi]]) * multv
                vy = plsc.load_gather(cky, [sel[wi]]) * multv
                vz = plsc.load_gather(ckz, [sel[wi]]) * multv
                ox[pl.ds(k * NUM_FPS + wi * L, L)] = vx
                oy[pl.ds(k * NUM_FPS + wi * L, L)] = vy
                oz[pl.ds(k * NUM_FPS + wi * L, L)] = vz

            cnt_vec = jnp.where(iota == k, countv, cnt_vec)

        blk = SEG_PER_TILE * NUM_FPS
        for d in range(3):
            pltpu.sync_copy(outv[d],
                            gp_hbm.at[pl.ds((d * NTILES + wid) * blk, blk)])
        cntv[...] = cnt_vec
        pltpu.sync_copy(cntv, cnt_hbm.at[pl.ds(wid * L, L)])

    gp_raw, cnt_raw = run(pts, labels)

    counts = cnt_raw.reshape(NTILES, L)[:, :SEG_PER_TILE].reshape(BS, SEG_NUM_ALL)
    masks = counts >= NUM_FPS
    gp = (
        gp_raw.reshape(3, BS, 4, SEG_PER_TILE, NUM_FPS)
        .transpose(1, 2, 3, 4, 0)
        .reshape(BS, SEG_NUM_ALL, NUM_FPS, 3)
    )
    return gp, masks
